# baseline (speedup 1.0000x reference)
import functools

import jax
import jax.numpy as jnp
from jax import lax
from jax.experimental import pallas as pl
from jax.experimental.pallas import tpu as pltpu
from jax.experimental.pallas import tpu_sc as plsc

N = 10000
E = 320000
D = 128
H = 64
G = 64

NC = 2
NS = 16
NW = NC * NS
CH = 128
NCH = 80
EPT = CH * NCH
EPAD = EPT * NW
NP = 10240
PAIR = NP // 2
RPT = NP // NS
HW = 16
XCH = RPT * 4 // CH
NBUF = 2
HBF = 2
CPB = 128


@functools.cache
def _sc_mesh():
  return plsc.VectorSubcoreMesh(
      core_axis_name="c", subcore_axis_name="s", num_cores=NC, num_subcores=NS)


def _hist_body(dst_hbm, eidx_hbm, out_hbm, acc, onesv, idxv, bufv, erows,
               eidxv, esem):
  c = lax.axis_index("c")
  s = lax.axis_index("s")

  def fill(i, carry):
    onesv[i, :] = jnp.full((HW,), 1.0, jnp.float32)
    return carry

  lax.fori_loop(0, CH, fill, 0)

  def zfill(i, carry):
    bufv[i, :] = jnp.zeros((HW,), jnp.float32)
    return carry

  lax.fori_loop(0, CPB, zfill, 0)
  for t in range(RPT // CPB):
    pltpu.sync_copy(bufv, acc.at[pl.ds(s * RPT + t * CPB, CPB)])
  plsc.subcore_barrier()
  pltpu.sync_copy(dst_hbm.at[c, s], idxv)

  def body(j, carry):
    pltpu.sync_copy(onesv, acc.at[idxv.at[j]], add=True)
    return carry

  lax.fori_loop(0, NCH, body, 0)
  plsc.subcore_barrier()
  pltpu.sync_copy(eidx_hbm.at[s], eidxv)
  for t in range(XCH):
    pltpu.async_copy(acc.at[eidxv.at[t]], erows.at[pl.ds(t * CH, CH)], esem)
  for t in range(XCH):
    pltpu.make_async_copy(acc.at[eidxv.at[0]], erows.at[pl.ds(0, CH)],
                          esem).wait()
  pltpu.sync_copy(erows, out_hbm.at[c, s])


@functools.cache
def _hist_call():
  return pl.kernel(
      _hist_body,
      out_type=jax.ShapeDtypeStruct((NC, NS, RPT * 4, HW), jnp.float32),
      mesh=_sc_mesh(),
      compiler_params=pltpu.CompilerParams(use_tc_tiling_on_sc=False),
      scratch_types=[
          pltpu.VMEM_SHARED((NP, HW), jnp.float32),
          pltpu.VMEM((CH, HW), jnp.float32),
          pltpu.VMEM((NCH, CH), jnp.int32),
          pltpu.VMEM((CPB, HW), jnp.float32),
          pltpu.VMEM((RPT * 4, HW), jnp.float32),
          pltpu.VMEM((XCH, CH), jnp.int32),
          pltpu.SemaphoreType.DMA,
      ],
  )


def _agg_body(src_hbm, dst_hbm, g_hbm, out_hbm, acc, gbuf, srcv, dstv, rows,
              gsems, ssems, stsem):
  c = lax.axis_index("c")
  s = lax.axis_index("s")
  zbuf = rows[0].at[pl.ds(0, CPB)]

  stage = pltpu.async_copy(
      g_hbm.at[pl.ds(s * RPT, RPT)], gbuf.at[pl.ds(s * RPT, RPT)], stsem)

  def fill(i, carry):
    for q in range(H // 16):
      rows[0][i, pl.ds(q * 16, 16)] = jnp.zeros((16,), jnp.float32)
    return carry

  lax.fori_loop(0, CPB, fill, 0)
  for t in range(RPT // CPB):
    pltpu.sync_copy(zbuf, acc.at[pl.ds(s * RPT + t * CPB, CPB)])
  stage.wait()
  plsc.subcore_barrier()

  pltpu.sync_copy(src_hbm.at[c, s], srcv)
  pltpu.sync_copy(dst_hbm.at[c, s], dstv)
  for b in range(NBUF):
    pltpu.async_copy(g_hbm.at[srcv.at[b]], rows[b], gsems[b])

  def body(t, carry):
    j0 = NBUF * t
    for b in range(NBUF):
      j = j0 + b
      pltpu.make_async_copy(gbuf.at[srcv.at[0]], rows[b], gsems[b]).wait()
      pltpu.async_copy(rows[b], acc.at[dstv.at[j]], ssems[b], add=True)

      @pl.when(j + NBUF < NCH)
      def _():
        pltpu.make_async_copy(rows[b], acc.at[dstv.at[0]], ssems[b]).wait()
        nxt = j + NBUF
        hbm_turn = lax.rem(nxt, 8) < HBF

        @pl.when(hbm_turn)
        def _():
          pltpu.async_copy(g_hbm.at[srcv.at[nxt]], rows[b], gsems[b])

        @pl.when(jnp.logical_not(hbm_turn))
        def _():
          pltpu.async_copy(gbuf.at[srcv.at[nxt]], rows[b], gsems[b])

    return carry

  lax.fori_loop(0, NCH // NBUF, body, 0)
  for b in range(NBUF):
    pltpu.make_async_copy(rows[b], acc.at[dstv.at[0]], ssems[b]).wait()

  plsc.subcore_barrier()
  for t in range(RPT // CPB):
    pltpu.sync_copy(acc.at[pl.ds(s * RPT + t * CPB, CPB)], zbuf)
    pltpu.sync_copy(zbuf, out_hbm.at[c, pl.ds(s * RPT + t * CPB, CPB)])


@functools.cache
def _agg_call():
  return pl.kernel(
      _agg_body,
      out_type=jax.ShapeDtypeStruct((NC, NP, H), jnp.float32),
      mesh=_sc_mesh(),
      compiler_params=pltpu.CompilerParams(use_tc_tiling_on_sc=False),
      scratch_types=[
          pltpu.VMEM_SHARED((NP, H), jnp.float32),
          pltpu.VMEM_SHARED((NP, H), jnp.float32),
          pltpu.VMEM((NCH, CH), jnp.int32),
          pltpu.VMEM((NCH, CH), jnp.int32),
          [pltpu.VMEM((CH, H), jnp.float32) for _ in range(NBUF)],
          [pltpu.SemaphoreType.DMA for _ in range(NBUF)],
          [pltpu.SemaphoreType.DMA for _ in range(NBUF)],
          pltpu.SemaphoreType.DMA,
      ],
  )


def _tc1_body(x_ref, w_ref, hist_ref, g_ref, dinv_ref):
  deg = hist_ref[0] + hist_ref[1] + 1.0
  dinv = lax.rsqrt(jnp.maximum(deg, 1.0))
  dinv_ref[...] = dinv
  g_ref[...] = dinv * jnp.dot(
      x_ref[...], w_ref[...], preferred_element_type=jnp.float32)


def _tc1_call(x2, wblk, hist):
  return pl.pallas_call(
      _tc1_body,
      out_shape=(
          jax.ShapeDtypeStruct((PAIR, 128), jnp.float32),
          jax.ShapeDtypeStruct((PAIR, 128), jnp.float32),
      ),
  )(x2, wblk, hist)


def _tc2_body(p_ref, g_ref, dinv_ref, b_ref, w_ref, o_ref):
  es = p_ref[0] + p_ref[1] + g_ref[...]
  h = jnp.maximum(dinv_ref[...] * es + b_ref[...], 0.0)
  o_ref[...] = dinv_ref[...] * jnp.dot(
      h, w_ref[...], preferred_element_type=jnp.float32)


def _tc2_call(p, g, dinv, b, wblk):
  return pl.pallas_call(
      _tc2_body,
      out_shape=jax.ShapeDtypeStruct((PAIR, 128), jnp.float32),
  )(p, g, dinv, b, wblk)


def _tc3_body(p_ref, g_ref, dinv_ref, b_ref, batch_ref, wfc_ref, bfc_ref,
              wout_ref, bout_ref, o_ref):
  h = jnp.maximum(
      dinv_ref[...] * (p_ref[0] + p_ref[1] + g_ref[...]) + b_ref[...], 0.0)
  seg = lax.broadcasted_iota(jnp.int32, (PAIR, 2 * G), 1)
  bl = batch_ref[...][:, 0:1]
  br = batch_ref[...][:, 1:2]
  msk = (seg < G).astype(jnp.float32)
  ohl = (bl == seg).astype(jnp.float32)
  ohr = (br == seg - G).astype(jnp.float32)
  oh = msk * ohl + (1.0 - msk) * ohr
  m = lax.dot_general(
      oh, h, (((0,), (0,)), ((), ())), preferred_element_type=jnp.float32)
  sums = lax.slice(m, (0, 0), (G, G)) + lax.slice(m, (G, G), (2 * G, 2 * G))
  cnt = lax.dot_general(
      oh, jnp.ones((PAIR, 1), jnp.float32), (((0,), (0,)), ((), ())),
      preferred_element_type=jnp.float32)
  counts = lax.slice(cnt, (0, 0), (G, 1)) + lax.slice(cnt, (G, 0), (2 * G, 1))
  pooled = sums / jnp.maximum(counts, 1.0)
  z = jnp.maximum(
      jnp.dot(pooled, wfc_ref[...], preferred_element_type=jnp.float32)
      + bfc_ref[...], 0.0)
  o_ref[...] = jnp.dot(
      z, wout_ref[...], preferred_element_type=jnp.float32) + bout_ref[...]


def _tc3_call(p, g, dinv, b, batch2, wfc, bfc, wout, bout):
  return pl.pallas_call(
      _tc3_body,
      out_shape=jax.ShapeDtypeStruct((G, 1), jnp.float32),
  )(p, g, dinv, b, batch2, wfc, bfc, wout, bout)


def _blockdiag2(w):
  k, m = w.shape
  z = jnp.zeros((2 * k, 2 * m), jnp.float32)
  return z.at[:k, :m].set(w).at[k:, m:].set(w)


def kernel(x, edge_index, batch, W1, b1, W2, b2, Wfc1, bfc1, Wout, bout):
  pad = jnp.full((EPAD - E,), N, jnp.int32)
  srcp = jnp.concatenate([edge_index[0].astype(jnp.int32), pad]).reshape(
      NC, NS, NCH, CH)
  dstp = jnp.concatenate([edge_index[1].astype(jnp.int32), pad]).reshape(
      NC, NS, NCH, CH)
  x2 = jnp.pad(x, ((0, NP - N), (0, 0))).reshape(PAIR, 2 * D)
  batch2 = jnp.pad(
      batch.astype(jnp.int32), (0, NP - N), constant_values=G).reshape(PAIR, 2)
  eidx = jnp.repeat(jnp.arange(NP, dtype=jnp.int32), 4).reshape(NS, XCH, CH)
  w1blk = _blockdiag2(W1)
  w2blk = _blockdiag2(W2)
  b1p = jnp.concatenate([b1, b1]).reshape(1, 2 * H)
  b2p = jnp.concatenate([b2, b2]).reshape(1, 2 * H)

  hist = _hist_call()(dstp, eidx).reshape(NC, PAIR, 128)
  g1, dinv = _tc1_call(x2, w1blk, hist)
  p1 = _agg_call()(srcp, dstp, g1.reshape(NP, H)).reshape(NC, PAIR, 128)
  g2 = _tc2_call(p1, g1, dinv, b1p, w2blk)
  p2 = _agg_call()(srcp, dstp, g2.reshape(NP, H)).reshape(NC, PAIR, 128)
  out = _tc3_call(p2, g2, dinv, b2p, batch2, Wfc1, bfc1.reshape(1, G), Wout,
                  bout.reshape(1, 1))
  return out


# speedup vs baseline: 1.2472x; 1.2472x over previous
import functools

import jax
import jax.numpy as jnp
from jax import lax
from jax.experimental import pallas as pl
from jax.experimental.pallas import tpu as pltpu
from jax.experimental.pallas import tpu_sc as plsc

N = 10000
E = 320000
D = 128
H = 64
G = 64

NC = 2
NS = 16
NW = NC * NS
CH = 128
NCH = 80
EPT = CH * NCH
EPAD = EPT * NW
NP = 10240
PAIR = NP // 2
RPT = NP // NS
HW = 16
XCH = RPT * 4 // CH
NBUF = 2
CPB = 128


@functools.cache
def _sc_mesh():
  return plsc.VectorSubcoreMesh(
      core_axis_name="c", subcore_axis_name="s", num_cores=NC, num_subcores=NS)


def _hist_body(dst_hbm, eidx_hbm, out_hbm, acc, onesv, idxv, bufv, erows,
               eidxv, esem):
  c = lax.axis_index("c")
  s = lax.axis_index("s")

  def fill(i, carry):
    onesv[i, :] = jnp.full((HW,), 1.0, jnp.float32)
    return carry

  lax.fori_loop(0, CH, fill, 0)

  def zfill(i, carry):
    bufv[i, :] = jnp.zeros((HW,), jnp.float32)
    return carry

  lax.fori_loop(0, CPB, zfill, 0)
  for t in range(RPT // CPB):
    pltpu.sync_copy(bufv, acc.at[pl.ds(s * RPT + t * CPB, CPB)])
  plsc.subcore_barrier()
  pltpu.sync_copy(dst_hbm.at[c, s], idxv)

  def body(j, carry):
    pltpu.sync_copy(onesv, acc.at[idxv.at[j]], add=True)
    return carry

  lax.fori_loop(0, NCH, body, 0)
  plsc.subcore_barrier()
  pltpu.sync_copy(eidx_hbm.at[s], eidxv)
  for t in range(XCH):
    pltpu.async_copy(acc.at[eidxv.at[t]], erows.at[pl.ds(t * CH, CH)], esem)
  for t in range(XCH):
    pltpu.make_async_copy(acc.at[eidxv.at[0]], erows.at[pl.ds(0, CH)],
                          esem).wait()
  pltpu.sync_copy(erows, out_hbm.at[c, s])


@functools.cache
def _hist_call():
  return pl.kernel(
      _hist_body,
      out_type=jax.ShapeDtypeStruct((NC, NS, RPT * 4, HW), jnp.float32),
      mesh=_sc_mesh(),
      compiler_params=pltpu.CompilerParams(use_tc_tiling_on_sc=False),
      scratch_types=[
          pltpu.VMEM_SHARED((NP, HW), jnp.float32),
          pltpu.VMEM((CH, HW), jnp.float32),
          pltpu.VMEM((NCH, CH), jnp.int32),
          pltpu.VMEM((CPB, HW), jnp.float32),
          pltpu.VMEM((RPT * 4, HW), jnp.float32),
          pltpu.VMEM((XCH, CH), jnp.int32),
          pltpu.SemaphoreType.DMA,
      ],
  )


def _agg_body(src_hbm, dst_hbm, g_hbm, out_hbm, acc, gbuf, srcv, dstv, rows,
              gsems, ssems, stsem):
  c = lax.axis_index("c")
  s = lax.axis_index("s")
  zbuf = rows[0].at[pl.ds(0, CPB)]

  stage = pltpu.async_copy(
      g_hbm.at[pl.ds(s * RPT, RPT)], gbuf.at[pl.ds(s * RPT, RPT)], stsem)

  def fill(i, carry):
    for q in range(H // 16):
      rows[0][i, pl.ds(q * 16, 16)] = jnp.zeros((16,), jnp.float32)
    return carry

  lax.fori_loop(0, CPB, fill, 0)
  for t in range(RPT // CPB):
    pltpu.sync_copy(zbuf, acc.at[pl.ds(s * RPT + t * CPB, CPB)])
  stage.wait()
  plsc.subcore_barrier()

  pltpu.sync_copy(src_hbm.at[c, s], srcv)
  pltpu.sync_copy(dst_hbm.at[c, s], dstv)
  for b in range(NBUF):
    pltpu.async_copy(gbuf.at[srcv.at[b]], rows[b], gsems[b])

  def body(t, carry):
    j0 = NBUF * t
    for b in range(NBUF):
      j = j0 + b
      pltpu.make_async_copy(gbuf.at[srcv.at[0]], rows[b], gsems[b]).wait()
      pltpu.async_copy(rows[b], acc.at[dstv.at[j]], ssems[b], add=True)

      @pl.when(j + NBUF < NCH)
      def _():
        pltpu.make_async_copy(rows[b], acc.at[dstv.at[0]], ssems[b]).wait()
        pltpu.async_copy(gbuf.at[srcv.at[j + NBUF]], rows[b], gsems[b])

    return carry

  lax.fori_loop(0, NCH // NBUF, body, 0)
  for b in range(NBUF):
    pltpu.make_async_copy(rows[b], acc.at[dstv.at[0]], ssems[b]).wait()

  plsc.subcore_barrier()
  for t in range(RPT // CPB):
    pltpu.sync_copy(acc.at[pl.ds(s * RPT + t * CPB, CPB)], zbuf)
    pltpu.sync_copy(zbuf, out_hbm.at[c, pl.ds(s * RPT + t * CPB, CPB)])


@functools.cache
def _agg_call():
  return pl.kernel(
      _agg_body,
      out_type=jax.ShapeDtypeStruct((NC, NP, H), jnp.float32),
      mesh=_sc_mesh(),
      compiler_params=pltpu.CompilerParams(use_tc_tiling_on_sc=False),
      scratch_types=[
          pltpu.VMEM_SHARED((NP, H), jnp.float32),
          pltpu.VMEM_SHARED((NP, H), jnp.float32),
          pltpu.VMEM((NCH, CH), jnp.int32),
          pltpu.VMEM((NCH, CH), jnp.int32),
          [pltpu.VMEM((CH, H), jnp.float32) for _ in range(NBUF)],
          [pltpu.SemaphoreType.DMA for _ in range(NBUF)],
          [pltpu.SemaphoreType.DMA for _ in range(NBUF)],
          pltpu.SemaphoreType.DMA,
      ],
  )


def _tc1_body(x_ref, w_ref, hist_ref, g_ref, dinv_ref):
  deg = hist_ref[0] + hist_ref[1] + 1.0
  dinv = lax.rsqrt(jnp.maximum(deg, 1.0))
  dinv_ref[...] = dinv
  g_ref[...] = dinv * jnp.dot(
      x_ref[...], w_ref[...], preferred_element_type=jnp.float32)


def _tc1_call(x2, wblk, hist):
  return pl.pallas_call(
      _tc1_body,
      out_shape=(
          jax.ShapeDtypeStruct((PAIR, 128), jnp.float32),
          jax.ShapeDtypeStruct((PAIR, 128), jnp.float32),
      ),
  )(x2, wblk, hist)


def _tc2_body(p_ref, g_ref, dinv_ref, b_ref, w_ref, o_ref):
  es = p_ref[0] + p_ref[1] + g_ref[...]
  h = jnp.maximum(dinv_ref[...] * es + b_ref[...], 0.0)
  o_ref[...] = dinv_ref[...] * jnp.dot(
      h, w_ref[...], preferred_element_type=jnp.float32)


def _tc2_call(p, g, dinv, b, wblk):
  return pl.pallas_call(
      _tc2_body,
      out_shape=jax.ShapeDtypeStruct((PAIR, 128), jnp.float32),
  )(p, g, dinv, b, wblk)


def _tc3_body(p_ref, g_ref, dinv_ref, b_ref, batch_ref, wfc_ref, bfc_ref,
              wout_ref, bout_ref, o_ref):
  h = jnp.maximum(
      dinv_ref[...] * (p_ref[0] + p_ref[1] + g_ref[...]) + b_ref[...], 0.0)
  seg = lax.broadcasted_iota(jnp.int32, (PAIR, 2 * G), 1)
  bl = batch_ref[...][:, 0:1]
  br = batch_ref[...][:, 1:2]
  msk = (seg < G).astype(jnp.float32)
  ohl = (bl == seg).astype(jnp.float32)
  ohr = (br == seg - G).astype(jnp.float32)
  oh = msk * ohl + (1.0 - msk) * ohr
  m = lax.dot_general(
      oh, h, (((0,), (0,)), ((), ())), preferred_element_type=jnp.float32)
  sums = lax.slice(m, (0, 0), (G, G)) + lax.slice(m, (G, G), (2 * G, 2 * G))
  cnt = lax.dot_general(
      oh, jnp.ones((PAIR, 1), jnp.float32), (((0,), (0,)), ((), ())),
      preferred_element_type=jnp.float32)
  counts = lax.slice(cnt, (0, 0), (G, 1)) + lax.slice(cnt, (G, 0), (2 * G, 1))
  pooled = sums / jnp.maximum(counts, 1.0)
  z = jnp.maximum(
      jnp.dot(pooled, wfc_ref[...], preferred_element_type=jnp.float32)
      + bfc_ref[...], 0.0)
  o_ref[...] = jnp.dot(
      z, wout_ref[...], preferred_element_type=jnp.float32) + bout_ref[...]


def _tc3_call(p, g, dinv, b, batch2, wfc, bfc, wout, bout):
  return pl.pallas_call(
      _tc3_body,
      out_shape=jax.ShapeDtypeStruct((G, 1), jnp.float32),
  )(p, g, dinv, b, batch2, wfc, bfc, wout, bout)


def _blockdiag2(w):
  k, m = w.shape
  z = jnp.zeros((2 * k, 2 * m), jnp.float32)
  return z.at[:k, :m].set(w).at[k:, m:].set(w)


def kernel(x, edge_index, batch, W1, b1, W2, b2, Wfc1, bfc1, Wout, bout):
  pad = jnp.full((EPAD - E,), N, jnp.int32)
  srcp = jnp.concatenate([edge_index[0].astype(jnp.int32), pad]).reshape(
      NC, NS, NCH, CH)
  dstp = jnp.concatenate([edge_index[1].astype(jnp.int32), pad]).reshape(
      NC, NS, NCH, CH)
  x2 = jnp.pad(x, ((0, NP - N), (0, 0))).reshape(PAIR, 2 * D)
  batch2 = jnp.pad(
      batch.astype(jnp.int32), (0, NP - N), constant_values=G).reshape(PAIR, 2)
  eidx = jnp.repeat(jnp.arange(NP, dtype=jnp.int32), 4).reshape(NS, XCH, CH)
  w1blk = _blockdiag2(W1)
  w2blk = _blockdiag2(W2)
  b1p = jnp.concatenate([b1, b1]).reshape(1, 2 * H)
  b2p = jnp.concatenate([b2, b2]).reshape(1, 2 * H)

  hist = _hist_call()(dstp, eidx).reshape(NC, PAIR, 128)
  g1, dinv = _tc1_call(x2, w1blk, hist)
  p1 = _agg_call()(srcp, dstp, g1.reshape(NP, H)).reshape(NC, PAIR, 128)
  g2 = _tc2_call(p1, g1, dinv, b1p, w2blk)
  p2 = _agg_call()(srcp, dstp, g2.reshape(NP, H)).reshape(NC, PAIR, 128)
  out = _tc3_call(p2, g2, dinv, b2p, batch2, Wfc1, bfc1.reshape(1, G), Wout,
                  bout.reshape(1, 1))
  return out
